# Initial kernel scaffold; baseline (speedup 1.0000x reference)
import jax
import jax.numpy as jnp
from jax.experimental import pallas as pl


def kernel(x, pos, edge_index, edge_attr, W1, b1, W2, b2, Wn1, bn1, Wn2, bn2, Wc1, bc1, Wc2, bc2):
    raise NotImplementedError("write your pallas kernel here")


# speedup vs baseline: 2.0301x; 2.0301x over previous
import functools

import jax
import jax.numpy as jnp
from jax import lax
from jax.experimental import pallas as pl
from jax.experimental.pallas import tpu as pltpu
from jax.experimental.pallas import tpu_sc as plsc

N = 10000
E = 320000
D = 128
H = 64
ED = 16
PW = 128
NPAD = 10240
NC = 2
NS = 16
NWK = NC * NS
EPW = E // NWK
CH = 80
NCH = EPW // CH
RPT = NPAD // NS

_HI = lax.Precision.HIGHEST


def _silu(v):
    return v * jax.nn.sigmoid(v)


def _prep_body(x_ref, pos_ref, w1a_ref, w1b_ref, b1_ref, pex_ref, qex_ref):
    x = x_ref[...]
    pos = pos_ref[...]
    p = lax.dot(x, w1a_ref[...], precision=_HI)
    q = lax.dot(x, w1b_ref[...], precision=_HI) + b1_ref[...]
    pad = jnp.zeros((x.shape[0], PW - H - 3), jnp.float32)
    pex_ref[...] = jnp.concatenate([p, pos, pad], axis=1)
    qex_ref[...] = jnp.concatenate([q, -pos, pad], axis=1)


def _prep(x, pos, w1a, w1b, b1r):
    bn = 2000
    return pl.pallas_call(
        _prep_body,
        grid=(N // bn,),
        in_specs=[
            pl.BlockSpec((bn, D), lambda i: (i, 0)),
            pl.BlockSpec((bn, 3), lambda i: (i, 0)),
            pl.BlockSpec((D, H), lambda i: (0, 0)),
            pl.BlockSpec((D, H), lambda i: (0, 0)),
            pl.BlockSpec((1, H), lambda i: (0, 0)),
        ],
        out_specs=[
            pl.BlockSpec((bn, PW), lambda i: (i, 0)),
            pl.BlockSpec((bn, PW), lambda i: (i, 0)),
        ],
        out_shape=[
            jax.ShapeDtypeStruct((N, PW), jnp.float32),
            jax.ShapeDtypeStruct((N, PW), jnp.float32),
        ],
    )(x, pos, w1a, w1b, b1r)


def _gather_body(pex, qex, row, col, out, idxr, idxc, bufp, bufq, semp, semq):
    wid = lax.axis_index("s") * NC + lax.axis_index("c")
    base = wid * EPW

    def chunk(i, carry):
        eb = base + i * CH
        pltpu.sync_copy(row.at[pl.ds(eb, CH)], idxr)
        pltpu.sync_copy(col.at[pl.ds(eb, CH)], idxc)
        cp = pltpu.async_copy(pex.at[idxr], bufp, semp)
        cq = pltpu.async_copy(qex.at[idxc], bufq, semq)
        cp.wait()
        cq.wait()

        def addrow(r, c2):
            for k in range(PW // 16):
                sl = pl.ds(k * 16, 16)
                bufp[r, sl] = bufp[r, sl] + bufq[r, sl]
            return c2

        lax.fori_loop(0, CH, addrow, 0)
        pltpu.sync_copy(bufp, out.at[pl.ds(eb, CH)])
        return carry

    lax.fori_loop(0, NCH, chunk, 0)


def _gather(pex, qex, row, col):
    mesh = plsc.VectorSubcoreMesh(core_axis_name="c", subcore_axis_name="s")
    f = functools.partial(
        pl.kernel,
        mesh=mesh,
        out_type=jax.ShapeDtypeStruct((E, PW), jnp.float32),
        scratch_types=[
            pltpu.VMEM((CH,), jnp.int32),
            pltpu.VMEM((CH,), jnp.int32),
            pltpu.VMEM((CH, PW), jnp.float32),
            pltpu.VMEM((CH, PW), jnp.float32),
            pltpu.SemaphoreType.DMA,
            pltpu.SemaphoreType.DMA,
        ],
    )(_gather_body)
    return f(pex, qex, row, col)


def _edge_body(s_ref, ea_ref, w1d_ref, wd_ref, w2_ref, b2_ref,
               wc1_ref, bc1_ref, wc2_ref, bc2_ref, m_ref):
    s = s_ref[...]
    pre0 = s[:, :H]
    diff = s[:, H:H + 3]
    dist = jnp.sqrt(jnp.sum(diff * diff, axis=1, keepdims=True))
    pre1 = (pre0 + dist * wd_ref[...]
            + lax.dot(ea_ref[...], w1d_ref[...], precision=_HI))
    h1 = _silu(pre1)
    msg = _silu(lax.dot(h1, w2_ref[...], precision=_HI) + b2_ref[...])
    t = _silu(lax.dot(msg, wc1_ref[...], precision=_HI) + bc1_ref[...])
    cw = jnp.sum(t * wc2_ref[...], axis=1, keepdims=True) + bc2_ref[...]
    pad = jnp.zeros((s.shape[0], PW - H - 3), jnp.float32)
    m_ref[...] = jnp.concatenate([msg, diff * cw, pad], axis=1)


def _edge_mlp(s, ea, w1d, wdr, w2, b2r, wc1, bc1r, wc2r, bc2r):
    be = 2000
    return pl.pallas_call(
        _edge_body,
        grid=(E // be,),
        in_specs=[
            pl.BlockSpec((be, PW), lambda i: (i, 0)),
            pl.BlockSpec((be, ED), lambda i: (i, 0)),
            pl.BlockSpec((ED, H), lambda i: (0, 0)),
            pl.BlockSpec((1, H), lambda i: (0, 0)),
            pl.BlockSpec((H, H), lambda i: (0, 0)),
            pl.BlockSpec((1, H), lambda i: (0, 0)),
            pl.BlockSpec((H, H), lambda i: (0, 0)),
            pl.BlockSpec((1, H), lambda i: (0, 0)),
            pl.BlockSpec((1, H), lambda i: (0, 0)),
            pl.BlockSpec((1, 1), lambda i: (0, 0)),
        ],
        out_specs=pl.BlockSpec((be, PW), lambda i: (i, 0)),
        out_shape=jax.ShapeDtypeStruct((E, PW), jnp.float32),
    )(s, ea, w1d, wdr, w2, b2r, wc1, bc1r, wc2r, bc2r)


NSC = NPAD // NC
ACC_R = NSC + 8
DUMP = NSC
RPT_D = NSC // NS
EPT = E // NS
NCH_D = EPT // CH


def _scatter_body(m, row, out, idxr, idx2, mbuf, zbuf, acc):
    cid = lax.axis_index("c")
    sid = lax.axis_index("s")
    lo = cid * NSC
    hi = lo + NSC
    zv = jnp.zeros((16,), jnp.float32)

    def zrow(r, c2):
        for k in range(PW // 16):
            zbuf[r, pl.ds(k * 16, 16)] = zv
        return c2

    lax.fori_loop(0, RPT_D, zrow, 0)
    pltpu.sync_copy(zbuf, acc.at[pl.ds(sid * RPT_D, RPT_D)])
    plsc.subcore_barrier()

    base = sid * EPT

    def chunk(i, carry):
        eb = base + i * CH
        pltpu.sync_copy(row.at[pl.ds(eb, CH)], idxr)
        pltpu.sync_copy(m.at[pl.ds(eb, CH)], mbuf)
        for k in range(CH // 16):
            sl = pl.ds(k * 16, 16)
            v = idxr[sl]
            inr = (v >= lo) & (v < hi)
            idx2[sl] = jnp.where(inr, v - lo, DUMP)
        pltpu.sync_copy(mbuf, acc.at[idx2], add=True)
        return carry

    lax.fori_loop(0, NCH_D, chunk, 0)
    plsc.subcore_barrier()
    pltpu.sync_copy(acc.at[pl.ds(sid * RPT_D, RPT_D)], zbuf)
    pltpu.sync_copy(zbuf, out.at[pl.ds(cid * NSC + sid * RPT_D, RPT_D)])


def _scatter(m, row):
    mesh = plsc.VectorSubcoreMesh(core_axis_name="c", subcore_axis_name="s")
    f = functools.partial(
        pl.kernel,
        mesh=mesh,
        out_type=jax.ShapeDtypeStruct((NPAD, PW), jnp.float32),
        scratch_types=[
            pltpu.VMEM((CH,), jnp.int32),
            pltpu.VMEM((CH,), jnp.int32),
            pltpu.VMEM((CH, PW), jnp.float32),
            pltpu.VMEM((RPT_D, PW), jnp.float32),
            pltpu.VMEM_SHARED((ACC_R, PW), jnp.float32),
        ],
    )(_scatter_body)
    return f(m, row)


def _node_body(x_ref, agg_ref, pos_ref, wn1a_ref, wn1b_ref, bn1_ref,
               wn2_ref, bn2_ref, xn_ref, pn_ref):
    agg = agg_ref[:, :H]
    coord = agg_ref[:, H:H + 3]
    h = _silu(lax.dot(x_ref[...], wn1a_ref[...], precision=_HI)
              + lax.dot(agg, wn1b_ref[...], precision=_HI) + bn1_ref[...])
    xn_ref[...] = lax.dot(h, wn2_ref[...], precision=_HI) + bn2_ref[...]
    pn_ref[...] = pos_ref[...] + coord


def _node_mlp(x, parts, pos, wn1a, wn1b, bn1r, wn2, bn2r):
    bn = 2000
    return pl.pallas_call(
        _node_body,
        grid=(N // bn,),
        in_specs=[
            pl.BlockSpec((bn, D), lambda i: (i, 0)),
            pl.BlockSpec((bn, PW), lambda i: (i, 0)),
            pl.BlockSpec((bn, 3), lambda i: (i, 0)),
            pl.BlockSpec((D, H), lambda i: (0, 0)),
            pl.BlockSpec((H, H), lambda i: (0, 0)),
            pl.BlockSpec((1, H), lambda i: (0, 0)),
            pl.BlockSpec((H, D), lambda i: (0, 0)),
            pl.BlockSpec((1, D), lambda i: (0, 0)),
        ],
        out_specs=[
            pl.BlockSpec((bn, D), lambda i: (i, 0)),
            pl.BlockSpec((bn, 3), lambda i: (i, 0)),
        ],
        out_shape=[
            jax.ShapeDtypeStruct((N, D), jnp.float32),
            jax.ShapeDtypeStruct((N, 3), jnp.float32),
        ],
    )(x, parts, pos, wn1a, wn1b, bn1r, wn2, bn2r)


def kernel(x, pos, edge_index, edge_attr, W1, b1, W2, b2,
           Wn1, bn1, Wn2, bn2, Wc1, bc1, Wc2, bc2):
    row = edge_index[0]
    col = edge_index[1]
    w1a = W1[:D]
    w1b = W1[D:2 * D]
    wdr = W1[2 * D:2 * D + 1]
    w1d = W1[2 * D + 1:]
    b1r = b1.reshape(1, H)
    b2r = b2.reshape(1, H)
    bc1r = bc1.reshape(1, H)
    wc2r = Wc2.reshape(1, H)
    bc2r = bc2.reshape(1, 1)
    bn1r = bn1.reshape(1, H)
    bn2r = bn2.reshape(1, D)
    wn1a = Wn1[:D]
    wn1b = Wn1[D:]

    pex, qex = _prep(x, pos, w1a, w1b, b1r)
    s = _gather(pex, qex, row, col)
    m = _edge_mlp(s, edge_attr, w1d, wdr, W2, b2r, Wc1, bc1r, wc2r, bc2r)
    parts = _scatter(m, row)
    return _node_mlp(x, parts, pos, wn1a, wn1b, bn1r, Wn2, bn2r)


# speedup vs baseline: 3.9321x; 1.9369x over previous
import functools

import jax
import jax.numpy as jnp
from jax import lax
from jax.experimental import pallas as pl
from jax.experimental.pallas import tpu as pltpu
from jax.experimental.pallas import tpu_sc as plsc

N = 10000
E = 320000
D = 128
H = 64
ED = 16
PW = 128
NPAD = 10240
NC = 2
NS = 16
NWK = NC * NS
EPW = E // NWK
CH = 80
NCH = EPW // CH
RPT = NPAD // NS

_HI = lax.Precision.HIGHEST


def _silu(v):
    return v * jax.nn.sigmoid(v)


def _prep_body(x_ref, pos_ref, w1a_ref, w1b_ref, b1_ref, pex_ref, qex_ref):
    x = x_ref[...]
    pos = pos_ref[...]
    p = lax.dot(x, w1a_ref[...], precision=_HI)
    q = lax.dot(x, w1b_ref[...], precision=_HI) + b1_ref[...]
    pad = jnp.zeros((x.shape[0], PW - H - 3), jnp.float32)
    pex_ref[...] = jnp.concatenate([p, pos, pad], axis=1)
    qex_ref[...] = jnp.concatenate([q, -pos, pad], axis=1)


def _prep(x, pos, w1a, w1b, b1r):
    bn = 2000
    return pl.pallas_call(
        _prep_body,
        grid=(N // bn,),
        in_specs=[
            pl.BlockSpec((bn, D), lambda i: (i, 0)),
            pl.BlockSpec((bn, 3), lambda i: (i, 0)),
            pl.BlockSpec((D, H), lambda i: (0, 0)),
            pl.BlockSpec((D, H), lambda i: (0, 0)),
            pl.BlockSpec((1, H), lambda i: (0, 0)),
        ],
        out_specs=[
            pl.BlockSpec((bn, PW), lambda i: (i, 0)),
            pl.BlockSpec((bn, PW), lambda i: (i, 0)),
        ],
        out_shape=[
            jax.ShapeDtypeStruct((N, PW), jnp.float32),
            jax.ShapeDtypeStruct((N, PW), jnp.float32),
        ],
    )(x, pos, w1a, w1b, b1r)


def _gather_body(pex, qex, row, col, out, idxr, idxc, bufp, bufq, semp, semq):
    wid = lax.axis_index("s") * NC + lax.axis_index("c")
    base = wid * EPW

    def chunk(i, carry):
        eb = base + i * CH
        pltpu.sync_copy(row.at[pl.ds(eb, CH)], idxr)
        pltpu.sync_copy(col.at[pl.ds(eb, CH)], idxc)
        cp = pltpu.async_copy(pex.at[idxr], bufp, semp)
        cq = pltpu.async_copy(qex.at[idxc], bufq, semq)
        cp.wait()
        cq.wait()

        def addrow(r, c2):
            for k in range(PW // 16):
                sl = pl.ds(k * 16, 16)
                bufp[r, sl] = bufp[r, sl] + bufq[r, sl]
            return c2

        lax.fori_loop(0, CH, addrow, 0)
        pltpu.sync_copy(bufp, out.at[pl.ds(eb, CH)])
        return carry

    lax.fori_loop(0, NCH, chunk, 0)


def _gather(pex, qex, row, col):
    mesh = plsc.VectorSubcoreMesh(core_axis_name="c", subcore_axis_name="s")
    f = functools.partial(
        pl.kernel,
        mesh=mesh,
        out_type=jax.ShapeDtypeStruct((E, PW), jnp.float32),
        scratch_types=[
            pltpu.VMEM((CH,), jnp.int32),
            pltpu.VMEM((CH,), jnp.int32),
            pltpu.VMEM((CH, PW), jnp.float32),
            pltpu.VMEM((CH, PW), jnp.float32),
            pltpu.SemaphoreType.DMA,
            pltpu.SemaphoreType.DMA,
        ],
    )(_gather_body)
    return f(pex, qex, row, col)


def _edge_body(s_ref, ea_ref, w1d_ref, wd_ref, w2_ref, b2_ref,
               wc1_ref, bc1_ref, wc2_ref, bc2_ref, msk_ref, ones_ref, m_ref):
    s = s_ref[...]
    sel = s * msk_ref[...]
    dist2 = lax.dot(sel * sel, ones_ref[...], precision=lax.Precision.DEFAULT)
    dist = jnp.sqrt(dist2)
    pre1 = (s + dist * wd_ref[...]
            + lax.dot(ea_ref[...], w1d_ref[...], precision=lax.Precision.DEFAULT))
    h1 = _silu(pre1)
    msg = _silu(lax.dot(h1, w2_ref[...], precision=lax.Precision.DEFAULT) + b2_ref[...])
    t = _silu(lax.dot(msg, wc1_ref[...], precision=lax.Precision.DEFAULT) + bc1_ref[...])
    cw = lax.dot(t, wc2_ref[...], precision=lax.Precision.DEFAULT) + bc2_ref[...]
    m_ref[...] = msg + sel * cw


def _edge_mlp(s, ea, w1dp, wdp, w2p, b2p, wc1p, bc1p, wc2c, bc2r, msk, onescol):
    be = 4000
    return pl.pallas_call(
        _edge_body,
        grid=(E // be,),
        in_specs=[
            pl.BlockSpec((be, PW), lambda i: (i, 0)),
            pl.BlockSpec((be, ED), lambda i: (i, 0)),
            pl.BlockSpec((ED, PW), lambda i: (0, 0)),
            pl.BlockSpec((1, PW), lambda i: (0, 0)),
            pl.BlockSpec((PW, PW), lambda i: (0, 0)),
            pl.BlockSpec((1, PW), lambda i: (0, 0)),
            pl.BlockSpec((PW, PW), lambda i: (0, 0)),
            pl.BlockSpec((1, PW), lambda i: (0, 0)),
            pl.BlockSpec((PW, 1), lambda i: (0, 0)),
            pl.BlockSpec((1, 1), lambda i: (0, 0)),
            pl.BlockSpec((1, PW), lambda i: (0, 0)),
            pl.BlockSpec((PW, 1), lambda i: (0, 0)),
        ],
        out_specs=pl.BlockSpec((be, PW), lambda i: (i, 0)),
        out_shape=jax.ShapeDtypeStruct((E, PW), jnp.float32),
    )(s, ea, w1dp, wdp, w2p, b2p, wc1p, bc1p, wc2c, bc2r, msk, onescol)


NSC = NPAD // NC
ACC_R = NSC + 8
DUMP = NSC
RPT_D = NSC // NS
EPT = E // NS
NCH_D = EPT // CH


def _scatter_body(m, row, out, idxr, idx2, mbuf, zbuf, acc):
    cid = lax.axis_index("c")
    sid = lax.axis_index("s")
    lo = cid * NSC
    hi = lo + NSC
    zv = jnp.zeros((16,), jnp.float32)

    def zrow(r, c2):
        for k in range(PW // 16):
            zbuf[r, pl.ds(k * 16, 16)] = zv
        return c2

    lax.fori_loop(0, RPT_D, zrow, 0)
    pltpu.sync_copy(zbuf, acc.at[pl.ds(sid * RPT_D, RPT_D)])
    plsc.subcore_barrier()

    base = sid * EPT

    def chunk(i, carry):
        eb = base + i * CH
        pltpu.sync_copy(row.at[pl.ds(eb, CH)], idxr)
        pltpu.sync_copy(m.at[pl.ds(eb, CH)], mbuf)
        for k in range(CH // 16):
            sl = pl.ds(k * 16, 16)
            v = idxr[sl]
            inr = (v >= lo) & (v < hi)
            idx2[sl] = jnp.where(inr, v - lo, DUMP)
        pltpu.sync_copy(mbuf, acc.at[idx2], add=True)
        return carry

    lax.fori_loop(0, NCH_D, chunk, 0)
    plsc.subcore_barrier()
    pltpu.sync_copy(acc.at[pl.ds(sid * RPT_D, RPT_D)], zbuf)
    pltpu.sync_copy(zbuf, out.at[pl.ds(cid * NSC + sid * RPT_D, RPT_D)])


def _scatter(m, row):
    mesh = plsc.VectorSubcoreMesh(core_axis_name="c", subcore_axis_name="s")
    f = functools.partial(
        pl.kernel,
        mesh=mesh,
        out_type=jax.ShapeDtypeStruct((NPAD, PW), jnp.float32),
        scratch_types=[
            pltpu.VMEM((CH,), jnp.int32),
            pltpu.VMEM((CH,), jnp.int32),
            pltpu.VMEM((CH, PW), jnp.float32),
            pltpu.VMEM((RPT_D, PW), jnp.float32),
            pltpu.VMEM_SHARED((ACC_R, PW), jnp.float32),
        ],
    )(_scatter_body)
    return f(m, row)


def _node_body(x_ref, agg_ref, pos_ref, wn1a_ref, wn1b_ref, bn1_ref,
               wn2_ref, bn2_ref, xn_ref, pn_ref):
    agg = agg_ref[:, :H]
    coord = agg_ref[:, H:H + 3]
    h = _silu(lax.dot(x_ref[...], wn1a_ref[...], precision=_HI)
              + lax.dot(agg, wn1b_ref[...], precision=_HI) + bn1_ref[...])
    xn_ref[...] = lax.dot(h, wn2_ref[...], precision=_HI) + bn2_ref[...]
    pn_ref[...] = pos_ref[...] + coord


def _node_mlp(x, parts, pos, wn1a, wn1b, bn1r, wn2, bn2r):
    bn = 2000
    return pl.pallas_call(
        _node_body,
        grid=(N // bn,),
        in_specs=[
            pl.BlockSpec((bn, D), lambda i: (i, 0)),
            pl.BlockSpec((bn, PW), lambda i: (i, 0)),
            pl.BlockSpec((bn, 3), lambda i: (i, 0)),
            pl.BlockSpec((D, H), lambda i: (0, 0)),
            pl.BlockSpec((H, H), lambda i: (0, 0)),
            pl.BlockSpec((1, H), lambda i: (0, 0)),
            pl.BlockSpec((H, D), lambda i: (0, 0)),
            pl.BlockSpec((1, D), lambda i: (0, 0)),
        ],
        out_specs=[
            pl.BlockSpec((bn, D), lambda i: (i, 0)),
            pl.BlockSpec((bn, 3), lambda i: (i, 0)),
        ],
        out_shape=[
            jax.ShapeDtypeStruct((N, D), jnp.float32),
            jax.ShapeDtypeStruct((N, 3), jnp.float32),
        ],
    )(x, parts, pos, wn1a, wn1b, bn1r, wn2, bn2r)


def kernel(x, pos, edge_index, edge_attr, W1, b1, W2, b2,
           Wn1, bn1, Wn2, bn2, Wc1, bc1, Wc2, bc2):
    row = edge_index[0]
    col = edge_index[1]
    w1a = W1[:D]
    w1b = W1[D:2 * D]
    b1r = b1.reshape(1, H)
    bn1r = bn1.reshape(1, H)
    bn2r = bn2.reshape(1, D)
    wn1a = Wn1[:D]
    wn1b = Wn1[D:]

    def padc(a):
        return jnp.pad(a, ((0, 0), (0, PW - a.shape[1])))

    def padr(a):
        return jnp.pad(a, ((0, PW - a.shape[0]), (0, 0)))

    wdp = padc(W1[2 * D:2 * D + 1])
    w1dp = padc(W1[2 * D + 1:])
    w2p = padr(padc(W2))
    b2p = padc(b2.reshape(1, H))
    wc1p = padr(padc(Wc1))
    bc1p = padc(bc1.reshape(1, H))
    wc2c = padr(Wc2)
    bc2r = bc2.reshape(1, 1)
    lane = jnp.arange(PW)
    msk = ((lane >= H) & (lane < H + 3)).astype(jnp.float32).reshape(1, PW)
    onescol = jnp.ones((PW, 1), jnp.float32)

    pex, qex = _prep(x, pos, w1a, w1b, b1r)
    s = _gather(pex, qex, row, col)
    m = _edge_mlp(s, edge_attr, w1dp, wdp, w2p, b2p, wc1p, bc1p, wc2c, bc2r,
                  msk, onescol)
    parts = _scatter(m, row)
    return _node_mlp(x, parts, pos, wn1a, wn1b, bn1r, Wn2, bn2r)


# speedup vs baseline: 6.0967x; 1.5505x over previous
import functools

import jax
import jax.numpy as jnp
from jax import lax
from jax.experimental import pallas as pl
from jax.experimental.pallas import tpu as pltpu
from jax.experimental.pallas import tpu_sc as plsc

N = 10000
E = 320000
D = 128
H = 64
ED = 16
PW = 128
NPAD = 10240
NC = 2
NS = 16
NWK = NC * NS
EPW = E // NWK
CH = 80
NCH = EPW // CH
RPT = NPAD // NS

_HI = lax.Precision.HIGHEST


def _silu(v):
    return v * jax.nn.sigmoid(v)


def _prep_body(x_ref, pos_ref, w1a_ref, w1b_ref, b1_ref, pex_ref, qex_ref):
    x = x_ref[...]
    pos = pos_ref[...]
    p = lax.dot(x, w1a_ref[...], precision=_HI)
    q = lax.dot(x, w1b_ref[...], precision=_HI) + b1_ref[...]
    pad = jnp.zeros((x.shape[0], PW - H - 3), jnp.float32)
    pex_ref[...] = jnp.concatenate([p, pos, pad], axis=1)
    qex_ref[...] = jnp.concatenate([q, -pos, pad], axis=1)


def _prep(x, pos, w1a, w1b, b1r):
    bn = 2000
    return pl.pallas_call(
        _prep_body,
        grid=(N // bn,),
        in_specs=[
            pl.BlockSpec((bn, D), lambda i: (i, 0)),
            pl.BlockSpec((bn, 3), lambda i: (i, 0)),
            pl.BlockSpec((D, H), lambda i: (0, 0)),
            pl.BlockSpec((D, H), lambda i: (0, 0)),
            pl.BlockSpec((1, H), lambda i: (0, 0)),
        ],
        out_specs=[
            pl.BlockSpec((bn, PW), lambda i: (i, 0)),
            pl.BlockSpec((bn, PW), lambda i: (i, 0)),
        ],
        out_shape=[
            jax.ShapeDtypeStruct((N, PW), jnp.float32),
            jax.ShapeDtypeStruct((N, PW), jnp.float32),
        ],
    )(x, pos, w1a, w1b, b1r)


def _gather_body(pex, qex, row2d, col2d, out,
                 idxr, idxc, bufp0, bufq0, bufp1, bufq1, outb0, outb1,
                 semg0, semg1, semw0, semw1):
    wid = lax.axis_index("s") * NC + lax.axis_index("c")
    base = wid * EPW
    bufp = (bufp0, bufp1)
    bufq = (bufq0, bufq1)
    outb = (outb0, outb1)
    semg = (semg0, semg1)
    semw = (semw0, semw1)

    pltpu.sync_copy(row2d.at[wid], idxr)
    pltpu.sync_copy(col2d.at[wid], idxc)

    def fire_gather(c, b):
        pltpu.async_copy(pex.at[idxr.at[c]], bufp[b], semg[b])
        pltpu.async_copy(qex.at[idxc.at[c]], bufq[b], semg[b])

    def wait_gather(b):
        pltpu.make_async_copy(pex.at[idxr.at[0]], bufp[b], semg[b]).wait()
        pltpu.make_async_copy(qex.at[idxc.at[0]], bufq[b], semg[b]).wait()

    def fire_write(c, b):
        pltpu.async_copy(outb[b], out.at[pl.ds(base + c * CH, CH)], semw[b])

    def wait_write(b):
        pltpu.make_async_copy(outb[b], out.at[pl.ds(base, CH)], semw[b]).wait()

    fire_gather(0, 0)
    fire_gather(1, 1)

    def body_one(c, b):
        wait_gather(b)

        @pl.when(c >= 2)
        def _():
            wait_write(b)

        def addrow(r, c2):
            for k in range(PW // 16):
                sl = pl.ds(k * 16, 16)
                outb[b][r, sl] = bufp[b][r, sl] + bufq[b][r, sl]
            return c2

        lax.fori_loop(0, CH, addrow, 0)
        fire_write(c, b)

        @pl.when(c + 2 < NCH)
        def _():
            fire_gather(c + 2, b)

    def outer(g, carry):
        body_one(2 * g, 0)

        @pl.when(2 * g + 1 < NCH)
        def _():
            body_one(2 * g + 1, 1)

        return carry

    lax.fori_loop(0, (NCH + 1) // 2, outer, 0)
    wait_write(0)
    wait_write(1)


def _gather(pex, qex, row2d, col2d):
    mesh = plsc.VectorSubcoreMesh(core_axis_name="c", subcore_axis_name="s")
    f = functools.partial(
        pl.kernel,
        mesh=mesh,
        out_type=jax.ShapeDtypeStruct((E, PW), jnp.float32),
        scratch_types=[
            pltpu.VMEM((NCH, CH), jnp.int32),
            pltpu.VMEM((NCH, CH), jnp.int32),
            pltpu.VMEM((CH, PW), jnp.float32),
            pltpu.VMEM((CH, PW), jnp.float32),
            pltpu.VMEM((CH, PW), jnp.float32),
            pltpu.VMEM((CH, PW), jnp.float32),
            pltpu.VMEM((CH, PW), jnp.float32),
            pltpu.VMEM((CH, PW), jnp.float32),
            pltpu.SemaphoreType.DMA,
            pltpu.SemaphoreType.DMA,
            pltpu.SemaphoreType.DMA,
            pltpu.SemaphoreType.DMA,
        ],
    )(_gather_body)
    return f(pex, qex, row2d, col2d)


def _edge_body(s_ref, ea_ref, w1d_ref, wd_ref, w2_ref, b2_ref,
               wc1_ref, bc1_ref, wc2_ref, bc2_ref, msk_ref, ones_ref, m_ref):
    s = s_ref[...]
    sel = s * msk_ref[...]
    dist2 = lax.dot(sel * sel, ones_ref[...], precision=lax.Precision.DEFAULT)
    dist = jnp.sqrt(dist2)
    pre1 = (s + dist * wd_ref[...]
            + lax.dot(ea_ref[...], w1d_ref[...], precision=lax.Precision.DEFAULT))
    h1 = _silu(pre1)
    msg = _silu(lax.dot(h1, w2_ref[...], precision=lax.Precision.DEFAULT) + b2_ref[...])
    t = _silu(lax.dot(msg, wc1_ref[...], precision=lax.Precision.DEFAULT) + bc1_ref[...])
    cw = lax.dot(t, wc2_ref[...], precision=lax.Precision.DEFAULT) + bc2_ref[...]
    m_ref[...] = msg + sel * cw


def _edge_mlp(s, ea, w1dp, wdp, w2p, b2p, wc1p, bc1p, wc2c, bc2r, msk, onescol):
    be = 4000
    return pl.pallas_call(
        _edge_body,
        grid=(E // be,),
        in_specs=[
            pl.BlockSpec((be, PW), lambda i: (i, 0)),
            pl.BlockSpec((be, ED), lambda i: (i, 0)),
            pl.BlockSpec((ED, PW), lambda i: (0, 0)),
            pl.BlockSpec((1, PW), lambda i: (0, 0)),
            pl.BlockSpec((PW, PW), lambda i: (0, 0)),
            pl.BlockSpec((1, PW), lambda i: (0, 0)),
            pl.BlockSpec((PW, PW), lambda i: (0, 0)),
            pl.BlockSpec((1, PW), lambda i: (0, 0)),
            pl.BlockSpec((PW, 1), lambda i: (0, 0)),
            pl.BlockSpec((1, 1), lambda i: (0, 0)),
            pl.BlockSpec((1, PW), lambda i: (0, 0)),
            pl.BlockSpec((PW, 1), lambda i: (0, 0)),
        ],
        out_specs=pl.BlockSpec((be, PW), lambda i: (i, 0)),
        out_shape=jax.ShapeDtypeStruct((E, PW), jnp.float32),
    )(s, ea, w1dp, wdp, w2p, b2p, wc1p, bc1p, wc2c, bc2r, msk, onescol)


NSC = NPAD // NC
ACC_R = NSC + 8
DUMP = NSC
RPT_D = NSC // NS
EPT = E // NS
NCH_D = EPT // CH


def _scatter_body(m, row2d, out, idxr, mbuf0, mbuf1, mbuf2, acc,
                  seml0, seml1, seml2, sems0, sems1, sems2):
    cid = lax.axis_index("c")
    sid = lax.axis_index("s")
    lo = cid * NSC
    hi = lo + NSC
    mbuf = (mbuf0, mbuf1, mbuf2)
    seml = (seml0, seml1, seml2)
    sems = (sems0, sems1, sems2)
    zv = jnp.zeros((16,), jnp.float32)

    pltpu.sync_copy(row2d.at[sid], idxr)

    def remap(r, c2):
        for k in range(CH // 16):
            sl = pl.ds(k * 16, 16)
            v = idxr[r, sl]
            inr = (v >= lo) & (v < hi)
            idxr[r, sl] = jnp.where(inr, v - lo, DUMP)
        return c2

    lax.fori_loop(0, NCH_D, remap, 0)

    def zrow(r, c2):
        for k in range(PW // 16):
            mbuf0[r, pl.ds(k * 16, 16)] = zv
        return c2

    lax.fori_loop(0, CH, zrow, 0)

    def zcopy(j, c2):
        pltpu.sync_copy(mbuf0, acc.at[pl.ds(sid * RPT_D + j * CH, CH)])
        return c2

    lax.fori_loop(0, RPT_D // CH, zcopy, 0)
    plsc.subcore_barrier()

    base = sid * EPT

    def fire_load(c, b):
        pltpu.async_copy(m.at[pl.ds(base + c * CH, CH)], mbuf[b], seml[b])

    def wait_load(b):
        pltpu.make_async_copy(m.at[pl.ds(base, CH)], mbuf[b], seml[b]).wait()

    def fire_scatter(c, b):
        pltpu.async_copy(mbuf[b], acc.at[idxr.at[c]], sems[b], add=True)

    def wait_scatter(b):
        pltpu.make_async_copy(mbuf[b], acc.at[idxr.at[0]], sems[b]).wait()

    fire_load(0, 0)
    fire_load(1, 1)

    def body_one(c, b, b2):
        wait_load(b)
        fire_scatter(c, b)

        @pl.when(c + 2 < NCH_D)
        def _():
            @pl.when(c >= 1)
            def _():
                wait_scatter(b2)

            fire_load(c + 2, b2)

    def outer(g, carry):
        c = 3 * g
        body_one(c, 0, 2)

        @pl.when(c + 1 < NCH_D)
        def _():
            body_one(c + 1, 1, 0)

        @pl.when(c + 2 < NCH_D)
        def _():
            body_one(c + 2, 2, 1)

        return carry

    lax.fori_loop(0, (NCH_D + 2) // 3, outer, 0)
    wait_scatter(0)
    wait_scatter(1)
    wait_scatter(2)
    plsc.subcore_barrier()

    def ocopy(j, c2):
        pltpu.sync_copy(acc.at[pl.ds(sid * RPT_D + j * CH, CH)], mbuf0)
        pltpu.sync_copy(mbuf0, out.at[pl.ds(cid * NSC + sid * RPT_D + j * CH, CH)])
        return c2

    lax.fori_loop(0, RPT_D // CH, ocopy, 0)


def _scatter(m, row2d):
    mesh = plsc.VectorSubcoreMesh(core_axis_name="c", subcore_axis_name="s")
    f = functools.partial(
        pl.kernel,
        mesh=mesh,
        out_type=jax.ShapeDtypeStruct((NPAD, PW), jnp.float32),
        scratch_types=[
            pltpu.VMEM((NCH_D, CH), jnp.int32),
            pltpu.VMEM((CH, PW), jnp.float32),
            pltpu.VMEM((CH, PW), jnp.float32),
            pltpu.VMEM((CH, PW), jnp.float32),
            pltpu.VMEM_SHARED((ACC_R, PW), jnp.float32),
            pltpu.SemaphoreType.DMA,
            pltpu.SemaphoreType.DMA,
            pltpu.SemaphoreType.DMA,
            pltpu.SemaphoreType.DMA,
            pltpu.SemaphoreType.DMA,
            pltpu.SemaphoreType.DMA,
        ],
    )(_scatter_body)
    return f(m, row2d)


def _node_body(x_ref, agg_ref, pos_ref, wn1a_ref, wn1b_ref, bn1_ref,
               wn2_ref, bn2_ref, xn_ref, pn_ref):
    agg = agg_ref[:, :H]
    coord = agg_ref[:, H:H + 3]
    h = _silu(lax.dot(x_ref[...], wn1a_ref[...], precision=_HI)
              + lax.dot(agg, wn1b_ref[...], precision=_HI) + bn1_ref[...])
    xn_ref[...] = lax.dot(h, wn2_ref[...], precision=_HI) + bn2_ref[...]
    pn_ref[...] = pos_ref[...] + coord


def _node_mlp(x, parts, pos, wn1a, wn1b, bn1r, wn2, bn2r):
    bn = 2000
    return pl.pallas_call(
        _node_body,
        grid=(N // bn,),
        in_specs=[
            pl.BlockSpec((bn, D), lambda i: (i, 0)),
            pl.BlockSpec((bn, PW), lambda i: (i, 0)),
            pl.BlockSpec((bn, 3), lambda i: (i, 0)),
            pl.BlockSpec((D, H), lambda i: (0, 0)),
            pl.BlockSpec((H, H), lambda i: (0, 0)),
            pl.BlockSpec((1, H), lambda i: (0, 0)),
            pl.BlockSpec((H, D), lambda i: (0, 0)),
            pl.BlockSpec((1, D), lambda i: (0, 0)),
        ],
        out_specs=[
            pl.BlockSpec((bn, D), lambda i: (i, 0)),
            pl.BlockSpec((bn, 3), lambda i: (i, 0)),
        ],
        out_shape=[
            jax.ShapeDtypeStruct((N, D), jnp.float32),
            jax.ShapeDtypeStruct((N, 3), jnp.float32),
        ],
    )(x, parts, pos, wn1a, wn1b, bn1r, wn2, bn2r)


def kernel(x, pos, edge_index, edge_attr, W1, b1, W2, b2,
           Wn1, bn1, Wn2, bn2, Wc1, bc1, Wc2, bc2):
    row3g = edge_index[0].reshape(NWK, NCH, CH)
    col3g = edge_index[1].reshape(NWK, NCH, CH)
    row3s = edge_index[0].reshape(NS, NCH_D, CH)
    w1a = W1[:D]
    w1b = W1[D:2 * D]
    b1r = b1.reshape(1, H)
    bn1r = bn1.reshape(1, H)
    bn2r = bn2.reshape(1, D)
    wn1a = Wn1[:D]
    wn1b = Wn1[D:]

    def padc(a):
        return jnp.pad(a, ((0, 0), (0, PW - a.shape[1])))

    def padr(a):
        return jnp.pad(a, ((0, PW - a.shape[0]), (0, 0)))

    wdp = padc(W1[2 * D:2 * D + 1])
    w1dp = padc(W1[2 * D + 1:])
    w2p = padr(padc(W2))
    b2p = padc(b2.reshape(1, H))
    wc1p = padr(padc(Wc1))
    bc1p = padc(bc1.reshape(1, H))
    wc2c = padr(Wc2)
    bc2r = bc2.reshape(1, 1)
    lane = jnp.arange(PW)
    msk = ((lane >= H) & (lane < H + 3)).astype(jnp.float32).reshape(1, PW)
    onescol = jnp.ones((PW, 1), jnp.float32)

    pex, qex = _prep(x, pos, w1a, w1b, b1r)
    s = _gather(pex, qex, row3g, col3g)
    m = _edge_mlp(s, edge_attr, w1dp, wdp, w2p, b2p, wc1p, bc1p, wc2c, bc2r,
                  msk, onescol)
    parts = _scatter(m, row3s)
    return _node_mlp(x, parts, pos, wn1a, wn1b, bn1r, Wn2, bn2r)


# speedup vs baseline: 6.9842x; 1.1456x over previous
import functools

import jax
import jax.numpy as jnp
from jax import lax
from jax.experimental import pallas as pl
from jax.experimental.pallas import tpu as pltpu
from jax.experimental.pallas import tpu_sc as plsc

N = 10000
E = 320000
D = 128
H = 64
ED = 16
PW = 128
NPAD = 10240
NC = 2
NS = 16
NWK = NC * NS
EPW = E // NWK
CH = 80
NCH = EPW // CH
RPT = NPAD // NS

_HI = lax.Precision.HIGHEST


def _silu(v):
    return v * jax.nn.sigmoid(v)


def _prep_body(x_ref, pos_ref, w1a_ref, w1b_ref, b1_ref, pex_ref, qex_ref):
    x = x_ref[...]
    pos = pos_ref[...]
    p = lax.dot(x, w1a_ref[...], precision=_HI)
    q = lax.dot(x, w1b_ref[...], precision=_HI) + b1_ref[...]
    pad = jnp.zeros((x.shape[0], PW - H - 3), jnp.float32)
    pex_ref[...] = jnp.concatenate([p, pos, pad], axis=1)
    qex_ref[...] = jnp.concatenate([q, -pos, pad], axis=1)


def _prep(x, pos, w1a, w1b, b1r):
    bn = 2000
    return pl.pallas_call(
        _prep_body,
        grid=(N // bn,),
        in_specs=[
            pl.BlockSpec((bn, D), lambda i: (i, 0)),
            pl.BlockSpec((bn, 3), lambda i: (i, 0)),
            pl.BlockSpec((D, H), lambda i: (0, 0)),
            pl.BlockSpec((D, H), lambda i: (0, 0)),
            pl.BlockSpec((1, H), lambda i: (0, 0)),
        ],
        out_specs=[
            pl.BlockSpec((bn, PW), lambda i: (i, 0)),
            pl.BlockSpec((bn, PW), lambda i: (i, 0)),
        ],
        out_shape=[
            jax.ShapeDtypeStruct((N, PW), jnp.float32),
            jax.ShapeDtypeStruct((N, PW), jnp.float32),
        ],
    )(x, pos, w1a, w1b, b1r)


def _make_gather_body(nch):
  def _gather_body(pex, qex, row2d, col2d, out,
                 idxr, idxc, bufp0, bufq0, bufp1, bufq1, outb0, outb1,
                 semg0, semg1, semw0, semw1):
    wid = lax.axis_index("s") * NC + lax.axis_index("c")
    base = wid * nch * CH
    bufp = (bufp0, bufp1)
    bufq = (bufq0, bufq1)
    outb = (outb0, outb1)
    semg = (semg0, semg1)
    semw = (semw0, semw1)

    pltpu.sync_copy(row2d.at[wid], idxr)
    pltpu.sync_copy(col2d.at[wid], idxc)

    def fire_gather(c, b):
        pltpu.async_copy(pex.at[idxr.at[c]], bufp[b], semg[b])
        pltpu.async_copy(qex.at[idxc.at[c]], bufq[b], semg[b])

    def wait_gather(b):
        pltpu.make_async_copy(pex.at[idxr.at[0]], bufp[b], semg[b]).wait()
        pltpu.make_async_copy(qex.at[idxc.at[0]], bufq[b], semg[b]).wait()

    def fire_write(c, b):
        pltpu.async_copy(outb[b], out.at[pl.ds(base + c * CH, CH)], semw[b])

    def wait_write(b):
        pltpu.make_async_copy(outb[b], out.at[pl.ds(base, CH)], semw[b]).wait()

    fire_gather(0, 0)
    fire_gather(1, 1)

    def body_one(c, b):
        wait_gather(b)

        @pl.when(c >= 2)
        def _():
            wait_write(b)

        def addrow(r, c2):
            for k in range(PW // 16):
                sl = pl.ds(k * 16, 16)
                outb[b][r, sl] = bufp[b][r, sl] + bufq[b][r, sl]
            return c2

        lax.fori_loop(0, CH, addrow, 0)
        fire_write(c, b)

        @pl.when(c + 2 < nch)
        def _():
            fire_gather(c + 2, b)

    def outer(g, carry):
        body_one(2 * g, 0)

        @pl.when(2 * g + 1 < nch)
        def _():
            body_one(2 * g + 1, 1)

        return carry

    lax.fori_loop(0, (nch + 1) // 2, outer, 0)
    wait_write(0)
    wait_write(1)
  return _gather_body


def _gather(pex, qex, row2d, col2d, nch):
    mesh = plsc.VectorSubcoreMesh(core_axis_name="c", subcore_axis_name="s")
    f = functools.partial(
        pl.kernel,
        mesh=mesh,
        out_type=jax.ShapeDtypeStruct((NWK * nch * CH, PW), jnp.float32),
        scratch_types=[
            pltpu.VMEM((nch, CH), jnp.int32),
            pltpu.VMEM((nch, CH), jnp.int32),
            pltpu.VMEM((CH, PW), jnp.float32),
            pltpu.VMEM((CH, PW), jnp.float32),
            pltpu.VMEM((CH, PW), jnp.float32),
            pltpu.VMEM((CH, PW), jnp.float32),
            pltpu.VMEM((CH, PW), jnp.float32),
            pltpu.VMEM((CH, PW), jnp.float32),
            pltpu.SemaphoreType.DMA,
            pltpu.SemaphoreType.DMA,
            pltpu.SemaphoreType.DMA,
            pltpu.SemaphoreType.DMA,
        ],
    )(_make_gather_body(nch))
    return f(pex, qex, row2d, col2d)


def _edge_body(s_ref, ea_ref, w1d_ref, wd_ref, w2_ref, b2_ref,
               wc1_ref, bc1_ref, wc2_ref, bc2_ref, msk_ref, ones_ref, m_ref):
    s = s_ref[...]
    sel = s * msk_ref[...]
    dist2 = lax.dot(sel * sel, ones_ref[...], precision=lax.Precision.DEFAULT)
    dist = jnp.sqrt(dist2)
    pre1 = (s + dist * wd_ref[...]
            + lax.dot(ea_ref[...], w1d_ref[...], precision=lax.Precision.DEFAULT))
    h1 = _silu(pre1)
    msg = _silu(lax.dot(h1, w2_ref[...], precision=lax.Precision.DEFAULT) + b2_ref[...])
    t = _silu(lax.dot(msg, wc1_ref[...], precision=lax.Precision.DEFAULT) + bc1_ref[...])
    cw = lax.dot(t, wc2_ref[...], precision=lax.Precision.DEFAULT) + bc2_ref[...]
    m_ref[...] = msg + sel * cw


def _edge_mlp(s, ea, w1dp, wdp, w2p, b2p, wc1p, bc1p, wc2c, bc2r, msk, onescol,
              rows, be):
    return pl.pallas_call(
        _edge_body,
        grid=(rows // be,),
        in_specs=[
            pl.BlockSpec((be, PW), lambda i: (i, 0)),
            pl.BlockSpec((be, ED), lambda i: (i, 0)),
            pl.BlockSpec((ED, PW), lambda i: (0, 0)),
            pl.BlockSpec((1, PW), lambda i: (0, 0)),
            pl.BlockSpec((PW, PW), lambda i: (0, 0)),
            pl.BlockSpec((1, PW), lambda i: (0, 0)),
            pl.BlockSpec((PW, PW), lambda i: (0, 0)),
            pl.BlockSpec((1, PW), lambda i: (0, 0)),
            pl.BlockSpec((PW, 1), lambda i: (0, 0)),
            pl.BlockSpec((1, 1), lambda i: (0, 0)),
            pl.BlockSpec((1, PW), lambda i: (0, 0)),
            pl.BlockSpec((PW, 1), lambda i: (0, 0)),
        ],
        out_specs=pl.BlockSpec((be, PW), lambda i: (i, 0)),
        out_shape=jax.ShapeDtypeStruct((rows, PW), jnp.float32),
    )(s, ea, w1dp, wdp, w2p, b2p, wc1p, bc1p, wc2c, bc2r, msk, onescol)


NSC = NPAD // NC
ACC_R = NSC + 8
DUMP = NSC
RPT_D = NSC // NS
EPT = E // NS
NCH_D = EPT // CH


def _make_scatter_body(nchd):
  def _scatter_body(m, row2d, out, idxr, mbuf0, mbuf1, mbuf2, acc,
                  seml0, seml1, seml2, sems0, sems1, sems2):
    cid = lax.axis_index("c")
    sid = lax.axis_index("s")
    lo = cid * NSC
    hi = lo + NSC
    mbuf = (mbuf0, mbuf1, mbuf2)
    seml = (seml0, seml1, seml2)
    sems = (sems0, sems1, sems2)
    zv = jnp.zeros((16,), jnp.float32)

    pltpu.sync_copy(row2d.at[sid], idxr)

    def remap(r, c2):
        for k in range(CH // 16):
            sl = pl.ds(k * 16, 16)
            v = idxr[r, sl]
            inr = (v >= lo) & (v < hi)
            idxr[r, sl] = jnp.where(inr, v - lo, DUMP)
        return c2

    lax.fori_loop(0, nchd, remap, 0)

    def zrow(r, c2):
        for k in range(PW // 16):
            mbuf0[r, pl.ds(k * 16, 16)] = zv
        return c2

    lax.fori_loop(0, CH, zrow, 0)

    def zcopy(j, c2):
        pltpu.sync_copy(mbuf0, acc.at[pl.ds(sid * RPT_D + j * CH, CH)])
        return c2

    lax.fori_loop(0, RPT_D // CH, zcopy, 0)
    plsc.subcore_barrier()

    base = sid * nchd * CH

    def fire_load(c, b):
        pltpu.async_copy(m.at[pl.ds(base + c * CH, CH)], mbuf[b], seml[b])

    def wait_load(b):
        pltpu.make_async_copy(m.at[pl.ds(base, CH)], mbuf[b], seml[b]).wait()

    def fire_scatter(c, b):
        pltpu.async_copy(mbuf[b], acc.at[idxr.at[c]], sems[b], add=True)

    def wait_scatter(b):
        pltpu.make_async_copy(mbuf[b], acc.at[idxr.at[0]], sems[b]).wait()

    fire_load(0, 0)
    fire_load(1, 1)

    def body_one(c, b, b2):
        wait_load(b)
        fire_scatter(c, b)

        @pl.when(c + 2 < nchd)
        def _():
            @pl.when(c >= 1)
            def _():
                wait_scatter(b2)

            fire_load(c + 2, b2)

    def outer(g, carry):
        c = 3 * g
        body_one(c, 0, 2)

        @pl.when(c + 1 < nchd)
        def _():
            body_one(c + 1, 1, 0)

        @pl.when(c + 2 < nchd)
        def _():
            body_one(c + 2, 2, 1)

        return carry

    lax.fori_loop(0, (nchd + 2) // 3, outer, 0)
    wait_scatter(0)
    wait_scatter(1)
    wait_scatter(2)
    plsc.subcore_barrier()

    def ocopy(j, c2):
        pltpu.sync_copy(acc.at[pl.ds(sid * RPT_D + j * CH, CH)], mbuf0)
        pltpu.sync_copy(mbuf0, out.at[pl.ds(cid * NSC + sid * RPT_D + j * CH, CH)])
        return c2

    lax.fori_loop(0, RPT_D // CH, ocopy, 0)
  return _scatter_body


def _scatter(m, row2d, nchd):
    mesh = plsc.VectorSubcoreMesh(core_axis_name="c", subcore_axis_name="s")
    f = functools.partial(
        pl.kernel,
        mesh=mesh,
        out_type=jax.ShapeDtypeStruct((NPAD, PW), jnp.float32),
        scratch_types=[
            pltpu.VMEM((nchd, CH), jnp.int32),
            pltpu.VMEM((CH, PW), jnp.float32),
            pltpu.VMEM((CH, PW), jnp.float32),
            pltpu.VMEM((CH, PW), jnp.float32),
            pltpu.VMEM_SHARED((ACC_R, PW), jnp.float32),
            pltpu.SemaphoreType.DMA,
            pltpu.SemaphoreType.DMA,
            pltpu.SemaphoreType.DMA,
            pltpu.SemaphoreType.DMA,
            pltpu.SemaphoreType.DMA,
            pltpu.SemaphoreType.DMA,
        ],
    )(_make_scatter_body(nchd))
    return f(m, row2d)


def _node_body(x_ref, agg_ref, agg2_ref, pos_ref, wn1a_ref, wn1b_ref, bn1_ref,
               wn2_ref, bn2_ref, xn_ref, pn_ref):
    aggf = agg_ref[...] + agg2_ref[...]
    agg = aggf[:, :H]
    coord = aggf[:, H:H + 3]
    h = _silu(lax.dot(x_ref[...], wn1a_ref[...], precision=_HI)
              + lax.dot(agg, wn1b_ref[...], precision=_HI) + bn1_ref[...])
    xn_ref[...] = lax.dot(h, wn2_ref[...], precision=_HI) + bn2_ref[...]
    pn_ref[...] = pos_ref[...] + coord


def _node_mlp(x, parts, parts2, pos, wn1a, wn1b, bn1r, wn2, bn2r):
    bn = 2000
    return pl.pallas_call(
        _node_body,
        grid=(N // bn,),
        in_specs=[
            pl.BlockSpec((bn, D), lambda i: (i, 0)),
            pl.BlockSpec((bn, PW), lambda i: (i, 0)),
            pl.BlockSpec((bn, PW), lambda i: (i, 0)),
            pl.BlockSpec((bn, 3), lambda i: (i, 0)),
            pl.BlockSpec((D, H), lambda i: (0, 0)),
            pl.BlockSpec((H, H), lambda i: (0, 0)),
            pl.BlockSpec((1, H), lambda i: (0, 0)),
            pl.BlockSpec((H, D), lambda i: (0, 0)),
            pl.BlockSpec((1, D), lambda i: (0, 0)),
        ],
        out_specs=[
            pl.BlockSpec((bn, D), lambda i: (i, 0)),
            pl.BlockSpec((bn, 3), lambda i: (i, 0)),
        ],
        out_shape=[
            jax.ShapeDtypeStruct((N, D), jnp.float32),
            jax.ShapeDtypeStruct((N, 3), jnp.float32),
        ],
    )(x, parts, parts2, pos, wn1a, wn1b, bn1r, wn2, bn2r)


def kernel(x, pos, edge_index, edge_attr, W1, b1, W2, b2,
           Wn1, bn1, Wn2, bn2, Wc1, bc1, Wc2, bc2):
    ncg0 = 62
    e0 = NWK * ncg0 * CH
    ncg1 = NCH - ncg0
    ncd0 = e0 // (NS * CH)
    ncd1 = (E - e0) // (NS * CH)
    row = edge_index[0]
    col = edge_index[1]
    r3g0 = row[:e0].reshape(NWK, ncg0, CH)
    c3g0 = col[:e0].reshape(NWK, ncg0, CH)
    r3g1 = row[e0:].reshape(NWK, ncg1, CH)
    c3g1 = col[e0:].reshape(NWK, ncg1, CH)
    r3s0 = row[:e0].reshape(NS, ncd0, CH)
    r3s1 = row[e0:].reshape(NS, ncd1, CH)
    w1a = W1[:D]
    w1b = W1[D:2 * D]
    b1r = b1.reshape(1, H)
    bn1r = bn1.reshape(1, H)
    bn2r = bn2.reshape(1, D)
    wn1a = Wn1[:D]
    wn1b = Wn1[D:]

    def padc(a):
        return jnp.pad(a, ((0, 0), (0, PW - a.shape[1])))

    def padr(a):
        return jnp.pad(a, ((0, PW - a.shape[0]), (0, 0)))

    wdp = padc(W1[2 * D:2 * D + 1])
    w1dp = padc(W1[2 * D + 1:])
    w2p = padr(padc(W2))
    b2p = padc(b2.reshape(1, H))
    wc1p = padr(padc(Wc1))
    bc1p = padc(bc1.reshape(1, H))
    wc2c = padr(Wc2)
    bc2r = bc2.reshape(1, 1)
    lane = jnp.arange(PW)
    msk = ((lane >= H) & (lane < H + 3)).astype(jnp.float32).reshape(1, PW)
    onescol = jnp.ones((PW, 1), jnp.float32)

    pex, qex = _prep(x, pos, w1a, w1b, b1r)
    s0 = _gather(pex, qex, r3g0, c3g0, ncg0)
    s1 = _gather(pex, qex, r3g1, c3g1, ncg1)
    m0 = _edge_mlp(s0, edge_attr[:e0], w1dp, wdp, w2p, b2p, wc1p, bc1p,
                   wc2c, bc2r, msk, onescol, e0, 3968)
    m1 = _edge_mlp(s1, edge_attr[e0:], w1dp, wdp, w2p, b2p, wc1p, bc1p,
                   wc2c, bc2r, msk, onescol, E - e0, 4032)
    agg0 = _scatter(m0, r3s0, ncd0)
    agg1 = _scatter(m1, r3s1, ncd1)
    return _node_mlp(x, agg0, agg1, pos, wn1a, wn1b, bn1r, Wn2, bn2r)


# speedup vs baseline: 7.4243x; 1.0630x over previous
import functools

import jax
import jax.numpy as jnp
from jax import lax
from jax.experimental import pallas as pl
from jax.experimental.pallas import tpu as pltpu
from jax.experimental.pallas import tpu_sc as plsc

N = 10000
E = 320000
D = 128
H = 64
ED = 16
PW = 128
NPAD = 10240
NC = 2
NS = 16
NWK = NC * NS
EPW = E // NWK
CH = 80
NCH = EPW // CH
RPT = NPAD // NS

_HI = lax.Precision.HIGHEST


def _silu(v):
    return v * jax.nn.sigmoid(v)


def _prep_body(x_ref, pos_ref, w1a_ref, w1b_ref, b1_ref, pex_ref, qex_ref):
    x = x_ref[...]
    pos = pos_ref[...]
    p = lax.dot(x, w1a_ref[...], precision=_HI)
    q = lax.dot(x, w1b_ref[...], precision=_HI) + b1_ref[...]
    pad = jnp.zeros((x.shape[0], PW - H - 3), jnp.float32)
    pex_ref[...] = jnp.concatenate([p, pos, pad], axis=1)
    qex_ref[...] = jnp.concatenate([q, -pos, pad], axis=1)


def _prep(x, pos, w1a, w1b, b1r):
    bn = 2000
    return pl.pallas_call(
        _prep_body,
        grid=(N // bn,),
        in_specs=[
            pl.BlockSpec((bn, D), lambda i: (i, 0)),
            pl.BlockSpec((bn, 3), lambda i: (i, 0)),
            pl.BlockSpec((D, H), lambda i: (0, 0)),
            pl.BlockSpec((D, H), lambda i: (0, 0)),
            pl.BlockSpec((1, H), lambda i: (0, 0)),
        ],
        out_specs=[
            pl.BlockSpec((bn, PW), lambda i: (i, 0)),
            pl.BlockSpec((bn, PW), lambda i: (i, 0)),
        ],
        out_shape=[
            jax.ShapeDtypeStruct((N, PW), jnp.float32),
            jax.ShapeDtypeStruct((N, PW), jnp.float32),
        ],
    )(x, pos, w1a, w1b, b1r)


def _make_gather_body(nch):
  def _gather_body(pex, qex, row2d, col2d, out,
                 idxr, idxc, bufp0, bufq0, bufp1, bufq1, outb0, outb1,
                 semg0, semg1, semw0, semw1):
    wid = lax.axis_index("s") * NC + lax.axis_index("c")
    base = wid * nch * CH
    bufp = (bufp0, bufp1)
    bufq = (bufq0, bufq1)
    outb = (outb0, outb1)
    semg = (semg0, semg1)
    semw = (semw0, semw1)

    pltpu.sync_copy(row2d.at[wid], idxr)
    pltpu.sync_copy(col2d.at[wid], idxc)

    def fire_gather(c, b):
        pltpu.async_copy(pex.at[idxr.at[c]], bufp[b], semg[b])
        pltpu.async_copy(qex.at[idxc.at[c]], bufq[b], semg[b])

    def wait_gather(b):
        pltpu.make_async_copy(pex.at[idxr.at[0]], bufp[b], semg[b]).wait()
        pltpu.make_async_copy(qex.at[idxc.at[0]], bufq[b], semg[b]).wait()

    def fire_write(c, b):
        pltpu.async_copy(outb[b], out.at[pl.ds(base + c * CH, CH)], semw[b])

    def wait_write(b):
        pltpu.make_async_copy(outb[b], out.at[pl.ds(base, CH)], semw[b]).wait()

    zv = jnp.zeros((16,), jnp.float32)

    def zpad(r, c2):
        for k in range(5, PW // 16):
            outb0[r, pl.ds(k * 16, 16)] = zv
            outb1[r, pl.ds(k * 16, 16)] = zv
        return c2

    lax.fori_loop(0, CH, zpad, 0)

    fire_gather(0, 0)
    fire_gather(1, 1)

    def body_one(c, b):
        wait_gather(b)

        @pl.when(c >= 2)
        def _():
            wait_write(b)

        def addrow(r, c2):
            for k in range(5):
                sl = pl.ds(k * 16, 16)
                outb[b][r, sl] = bufp[b][r, sl] + bufq[b][r, sl]
            return c2

        lax.fori_loop(0, CH, addrow, 0)
        fire_write(c, b)

        @pl.when(c + 2 < nch)
        def _():
            fire_gather(c + 2, b)

    def outer(g, carry):
        body_one(2 * g, 0)

        @pl.when(2 * g + 1 < nch)
        def _():
            body_one(2 * g + 1, 1)

        return carry

    lax.fori_loop(0, (nch + 1) // 2, outer, 0)
    wait_write(0)
    wait_write(1)
  return _gather_body


def _gather(pex, qex, row2d, col2d, nch):
    mesh = plsc.VectorSubcoreMesh(core_axis_name="c", subcore_axis_name="s")
    f = functools.partial(
        pl.kernel,
        mesh=mesh,
        out_type=jax.ShapeDtypeStruct((NWK * nch * CH, PW), jnp.float32),
        scratch_types=[
            pltpu.VMEM((nch, CH), jnp.int32),
            pltpu.VMEM((nch, CH), jnp.int32),
            pltpu.VMEM((CH, PW), jnp.float32),
            pltpu.VMEM((CH, PW), jnp.float32),
            pltpu.VMEM((CH, PW), jnp.float32),
            pltpu.VMEM((CH, PW), jnp.float32),
            pltpu.VMEM((CH, PW), jnp.float32),
            pltpu.VMEM((CH, PW), jnp.float32),
            pltpu.SemaphoreType.DMA,
            pltpu.SemaphoreType.DMA,
            pltpu.SemaphoreType.DMA,
            pltpu.SemaphoreType.DMA,
        ],
    )(_make_gather_body(nch))
    return f(pex, qex, row2d, col2d)


def _edge_body(s_ref, ea_ref, w1d_ref, wd_ref, w2_ref, b2_ref,
               wc1_ref, bc1_ref, wc2_ref, bc2_ref, msk_ref, msk0_ref, ones_ref,
               m_ref):
    s = s_ref[...]
    zero = jnp.zeros_like(s)
    sel = jnp.where(msk_ref[...] > 0.5, s, zero)
    dist2 = lax.dot(sel * sel, ones_ref[...], precision=lax.Precision.DEFAULT)
    dist = jnp.sqrt(dist2)
    pre1 = (jnp.where(msk0_ref[...] > 0.5, s, zero) + dist * wd_ref[...]
            + lax.dot(ea_ref[...], w1d_ref[...], precision=lax.Precision.DEFAULT))
    h1 = _silu(pre1)
    msg = _silu(lax.dot(h1, w2_ref[...], precision=lax.Precision.DEFAULT) + b2_ref[...])
    t = _silu(lax.dot(msg, wc1_ref[...], precision=lax.Precision.DEFAULT) + bc1_ref[...])
    cw = lax.dot(t, wc2_ref[...], precision=lax.Precision.DEFAULT) + bc2_ref[...]
    m_ref[...] = msg + sel * cw


def _edge_mlp(s, ea, w1dp, wdp, w2p, b2p, wc1p, bc1p, wc2c, bc2r, msk, msk0,
              onescol, rows, be):
    return pl.pallas_call(
        _edge_body,
        grid=(rows // be,),
        in_specs=[
            pl.BlockSpec((be, PW), lambda i: (i, 0)),
            pl.BlockSpec((be, ED), lambda i: (i, 0)),
            pl.BlockSpec((ED, PW), lambda i: (0, 0)),
            pl.BlockSpec((1, PW), lambda i: (0, 0)),
            pl.BlockSpec((PW, PW), lambda i: (0, 0)),
            pl.BlockSpec((1, PW), lambda i: (0, 0)),
            pl.BlockSpec((PW, PW), lambda i: (0, 0)),
            pl.BlockSpec((1, PW), lambda i: (0, 0)),
            pl.BlockSpec((PW, 1), lambda i: (0, 0)),
            pl.BlockSpec((1, 1), lambda i: (0, 0)),
            pl.BlockSpec((1, PW), lambda i: (0, 0)),
            pl.BlockSpec((1, PW), lambda i: (0, 0)),
            pl.BlockSpec((PW, 1), lambda i: (0, 0)),
        ],
        out_specs=pl.BlockSpec((be, PW), lambda i: (i, 0)),
        out_shape=jax.ShapeDtypeStruct((rows, PW), jnp.float32),
    )(s, ea, w1dp, wdp, w2p, b2p, wc1p, bc1p, wc2c, bc2r, msk, msk0, onescol)


NSC = NPAD // NC
ACC_R = NSC + 8
DUMP = NSC
RPT_D = NSC // NS
EPT = E // NS
NCH_D = EPT // CH


def _make_scatter_body(nchd):
  def _scatter_body(m, row2d, out, idxr, mbuf0, mbuf1, mbuf2, mbuf3, acc,
                  seml0, seml1, seml2, seml3, sems0, sems1, sems2, sems3):
    cid = lax.axis_index("c")
    sid = lax.axis_index("s")
    lo = cid * NSC
    hi = lo + NSC
    mbuf = (mbuf0, mbuf1, mbuf2, mbuf3)
    seml = (seml0, seml1, seml2, seml3)
    sems = (sems0, sems1, sems2, sems3)
    zv = jnp.zeros((16,), jnp.float32)

    pltpu.sync_copy(row2d.at[sid], idxr)

    def remap(r, c2):
        for k in range(CH // 16):
            sl = pl.ds(k * 16, 16)
            v = idxr[r, sl]
            inr = (v >= lo) & (v < hi)
            idxr[r, sl] = jnp.where(inr, v - lo, DUMP)
        return c2

    lax.fori_loop(0, nchd, remap, 0)

    def zrow(r, c2):
        for k in range(PW // 16):
            mbuf0[r, pl.ds(k * 16, 16)] = zv
        return c2

    lax.fori_loop(0, CH, zrow, 0)

    def zcopy(j, c2):
        pltpu.sync_copy(mbuf0, acc.at[pl.ds(sid * RPT_D + j * CH, CH)])
        return c2

    lax.fori_loop(0, RPT_D // CH, zcopy, 0)
    plsc.subcore_barrier()

    base = sid * nchd * CH

    def fire_load(c, b):
        pltpu.async_copy(m.at[pl.ds(base + c * CH, CH)], mbuf[b], seml[b])

    def wait_load(b):
        pltpu.make_async_copy(m.at[pl.ds(base, CH)], mbuf[b], seml[b]).wait()

    def fire_scatter(c, b):
        pltpu.async_copy(mbuf[b], acc.at[idxr.at[c]], sems[b], add=True)

    def wait_scatter(b):
        pltpu.make_async_copy(mbuf[b], acc.at[idxr.at[0]], sems[b]).wait()

    fire_load(0, 0)
    fire_load(1, 1)

    def body_one(c, b, b2):
        wait_load(b)
        fire_scatter(c, b)

        @pl.when(c + 2 < nchd)
        def _():
            @pl.when(c >= 2)
            def _():
                wait_scatter(b2)

            fire_load(c + 2, b2)

    def outer(g, carry):
        c = 4 * g
        body_one(c, 0, 2)

        @pl.when(c + 1 < nchd)
        def _():
            body_one(c + 1, 1, 3)

        @pl.when(c + 2 < nchd)
        def _():
            body_one(c + 2, 2, 0)

        @pl.when(c + 3 < nchd)
        def _():
            body_one(c + 3, 3, 1)

        return carry

    lax.fori_loop(0, (nchd + 3) // 4, outer, 0)
    wait_scatter(0)
    wait_scatter(1)
    wait_scatter(2)
    wait_scatter(3)
    plsc.subcore_barrier()

    def ocopy(j, c2):
        pltpu.sync_copy(acc.at[pl.ds(sid * RPT_D + j * CH, CH)], mbuf0)
        pltpu.sync_copy(mbuf0, out.at[pl.ds(cid * NSC + sid * RPT_D + j * CH, CH)])
        return c2

    lax.fori_loop(0, RPT_D // CH, ocopy, 0)
  return _scatter_body


def _scatter(m, row2d, nchd):
    mesh = plsc.VectorSubcoreMesh(core_axis_name="c", subcore_axis_name="s")
    f = functools.partial(
        pl.kernel,
        mesh=mesh,
        out_type=jax.ShapeDtypeStruct((NPAD, PW), jnp.float32),
        scratch_types=[
            pltpu.VMEM((nchd, CH), jnp.int32),
            pltpu.VMEM((CH, PW), jnp.float32),
            pltpu.VMEM((CH, PW), jnp.float32),
            pltpu.VMEM((CH, PW), jnp.float32),
            pltpu.VMEM((CH, PW), jnp.float32),
            pltpu.VMEM_SHARED((ACC_R, PW), jnp.float32),
            pltpu.SemaphoreType.DMA,
            pltpu.SemaphoreType.DMA,
            pltpu.SemaphoreType.DMA,
            pltpu.SemaphoreType.DMA,
            pltpu.SemaphoreType.DMA,
            pltpu.SemaphoreType.DMA,
            pltpu.SemaphoreType.DMA,
            pltpu.SemaphoreType.DMA,
        ],
    )(_make_scatter_body(nchd))
    return f(m, row2d)


def _node_body(x_ref, agg_ref, agg2_ref, agg3_ref, agg4_ref, pos_ref,
               wn1a_ref, wn1b_ref, bn1_ref, wn2_ref, bn2_ref, xn_ref, pn_ref):
    aggf = ((agg_ref[...] + agg2_ref[...])
            + (agg3_ref[...] + agg4_ref[...]))
    agg = aggf[:, :H]
    coord = aggf[:, H:H + 3]
    h = _silu(lax.dot(x_ref[...], wn1a_ref[...], precision=_HI)
              + lax.dot(agg, wn1b_ref[...], precision=_HI) + bn1_ref[...])
    xn_ref[...] = lax.dot(h, wn2_ref[...], precision=_HI) + bn2_ref[...]
    pn_ref[...] = pos_ref[...] + coord


def _node_mlp(x, parts, parts2, parts3, parts4, pos, wn1a, wn1b, bn1r,
              wn2, bn2r):
    bn = 2000
    return pl.pallas_call(
        _node_body,
        grid=(N // bn,),
        in_specs=[
            pl.BlockSpec((bn, D), lambda i: (i, 0)),
            pl.BlockSpec((bn, PW), lambda i: (i, 0)),
            pl.BlockSpec((bn, PW), lambda i: (i, 0)),
            pl.BlockSpec((bn, PW), lambda i: (i, 0)),
            pl.BlockSpec((bn, PW), lambda i: (i, 0)),
            pl.BlockSpec((bn, 3), lambda i: (i, 0)),
            pl.BlockSpec((D, H), lambda i: (0, 0)),
            pl.BlockSpec((H, H), lambda i: (0, 0)),
            pl.BlockSpec((1, H), lambda i: (0, 0)),
            pl.BlockSpec((H, D), lambda i: (0, 0)),
            pl.BlockSpec((1, D), lambda i: (0, 0)),
        ],
        out_specs=[
            pl.BlockSpec((bn, D), lambda i: (i, 0)),
            pl.BlockSpec((bn, 3), lambda i: (i, 0)),
        ],
        out_shape=[
            jax.ShapeDtypeStruct((N, D), jnp.float32),
            jax.ShapeDtypeStruct((N, 3), jnp.float32),
        ],
    )(x, parts, parts2, parts3, parts4, pos, wn1a, wn1b, bn1r, wn2, bn2r)


def kernel(x, pos, edge_index, edge_attr, W1, b1, W2, b2,
           Wn1, bn1, Wn2, bn2, Wc1, bc1, Wc2, bc2):
    ncgs = (31, 31, 31, 32)
    row = edge_index[0]
    col = edge_index[1]
    sl_edges = [NWK * g * CH for g in ncgs]
    bounds = [0]
    for n_e in sl_edges:
        bounds.append(bounds[-1] + n_e)
    r3g, c3g, r3s, ncds = [], [], [], []
    for i, g in enumerate(ncgs):
        a, b = bounds[i], bounds[i + 1]
        r3g.append(row[a:b].reshape(NWK, g, CH))
        c3g.append(col[a:b].reshape(NWK, g, CH))
        nd = (b - a) // (NS * CH)
        ncds.append(nd)
        r3s.append(row[a:b].reshape(NS, nd, CH))
    w1a = W1[:D]
    w1b = W1[D:2 * D]
    b1r = b1.reshape(1, H)
    bn1r = bn1.reshape(1, H)
    bn2r = bn2.reshape(1, D)
    wn1a = Wn1[:D]
    wn1b = Wn1[D:]

    def padc(a):
        return jnp.pad(a, ((0, 0), (0, PW - a.shape[1])))

    def padr(a):
        return jnp.pad(a, ((0, PW - a.shape[0]), (0, 0)))

    wdp = padc(W1[2 * D:2 * D + 1])
    w1dp = padc(W1[2 * D + 1:])
    w2p = padr(padc(W2))
    b2p = padc(b2.reshape(1, H))
    wc1p = padr(padc(Wc1))
    bc1p = padc(bc1.reshape(1, H))
    wc2c = padr(Wc2)
    bc2r = bc2.reshape(1, 1)
    lane = jnp.arange(PW)
    msk = ((lane >= H) & (lane < H + 3)).astype(jnp.float32).reshape(1, PW)
    msk0 = (lane < H).astype(jnp.float32).reshape(1, PW)
    onescol = jnp.ones((PW, 1), jnp.float32)

    pex, qex = _prep(x, pos, w1a, w1b, b1r)
    aggs = []
    for i, g in enumerate(ncgs):
        a, b = bounds[i], bounds[i + 1]
        s_i = _gather(pex, qex, r3g[i], c3g[i], g)
        m_i = _edge_mlp(s_i, edge_attr[a:b], w1dp, wdp, w2p, b2p, wc1p, bc1p,
                        wc2c, bc2r, msk, msk0, onescol, b - a, (b - a) // 20)
        aggs.append(_scatter(m_i, r3s[i], ncds[i]))
    return _node_mlp(x, aggs[0], aggs[1], aggs[2], aggs[3], pos,
                     wn1a, wn1b, bn1r, Wn2, bn2r)


# speedup vs baseline: 7.4707x; 1.0063x over previous
import functools

import jax
import jax.numpy as jnp
from jax import lax
from jax.experimental import pallas as pl
from jax.experimental.pallas import tpu as pltpu
from jax.experimental.pallas import tpu_sc as plsc

N = 10000
E = 320000
D = 128
H = 64
ED = 16
PW = 128
NPAD = 10240
NC = 2
NS = 16
NWK = NC * NS
EPW = E // NWK
CH = 80
NCH = EPW // CH
RPT = NPAD // NS

_HI = lax.Precision.HIGHEST


def _silu(v):
    return v * jax.nn.sigmoid(v)


def _prep_body(x_ref, pos_ref, w1a_ref, w1b_ref, b1_ref, pex_ref, qex_ref):
    x = x_ref[...]
    pos = pos_ref[...]
    p = lax.dot(x, w1a_ref[...], precision=_HI)
    q = lax.dot(x, w1b_ref[...], precision=_HI) + b1_ref[...]
    pad = jnp.zeros((x.shape[0], PW - H - 3), jnp.float32)
    pex_ref[...] = jnp.concatenate([p, pos, pad], axis=1)
    qex_ref[...] = jnp.concatenate([q, -pos, pad], axis=1)


def _prep(x, pos, w1a, w1b, b1r):
    bn = 2000
    return pl.pallas_call(
        _prep_body,
        grid=(N // bn,),
        in_specs=[
            pl.BlockSpec((bn, D), lambda i: (i, 0)),
            pl.BlockSpec((bn, 3), lambda i: (i, 0)),
            pl.BlockSpec((D, H), lambda i: (0, 0)),
            pl.BlockSpec((D, H), lambda i: (0, 0)),
            pl.BlockSpec((1, H), lambda i: (0, 0)),
        ],
        out_specs=[
            pl.BlockSpec((bn, PW), lambda i: (i, 0)),
            pl.BlockSpec((bn, PW), lambda i: (i, 0)),
        ],
        out_shape=[
            jax.ShapeDtypeStruct((N, PW), jnp.float32),
            jax.ShapeDtypeStruct((N, PW), jnp.float32),
        ],
    )(x, pos, w1a, w1b, b1r)


def _make_gather_body(nch):
  def _gather_body(pex, qex, row2d, col2d, out,
                 idxr, idxc, bufp0, bufq0, bufp1, bufq1, bufp2, bufq2,
                 outb0, outb1, outb2,
                 semg0, semg1, semg2, semw0, semw1, semw2):
    wid = lax.axis_index("s") * NC + lax.axis_index("c")
    base = wid * nch * CH
    bufp = (bufp0, bufp1, bufp2)
    bufq = (bufq0, bufq1, bufq2)
    outb = (outb0, outb1, outb2)
    semg = (semg0, semg1, semg2)
    semw = (semw0, semw1, semw2)

    pltpu.sync_copy(row2d.at[wid], idxr)
    pltpu.sync_copy(col2d.at[wid], idxc)

    def fire_gather(c, b):
        pltpu.async_copy(pex.at[idxr.at[c]], bufp[b], semg[b])
        pltpu.async_copy(qex.at[idxc.at[c]], bufq[b], semg[b])

    def wait_gather(b):
        pltpu.make_async_copy(pex.at[idxr.at[0]], bufp[b], semg[b]).wait()
        pltpu.make_async_copy(qex.at[idxc.at[0]], bufq[b], semg[b]).wait()

    def fire_write(c, b):
        pltpu.async_copy(outb[b], out.at[pl.ds(base + c * CH, CH)], semw[b])

    def wait_write(b):
        pltpu.make_async_copy(outb[b], out.at[pl.ds(base, CH)], semw[b]).wait()

    zv = jnp.zeros((16,), jnp.float32)

    def zpad(r, c2):
        for k in range(5, PW // 16):
            outb0[r, pl.ds(k * 16, 16)] = zv
            outb1[r, pl.ds(k * 16, 16)] = zv
            outb2[r, pl.ds(k * 16, 16)] = zv
        return c2

    lax.fori_loop(0, CH, zpad, 0)

    fire_gather(0, 0)
    fire_gather(1, 1)
    fire_gather(2, 2)

    def body_one(c, b):
        wait_gather(b)

        @pl.when(c >= 3)
        def _():
            wait_write(b)

        def addrow(r, c2):
            for k in range(5):
                sl = pl.ds(k * 16, 16)
                outb[b][r, sl] = bufp[b][r, sl] + bufq[b][r, sl]
            return c2

        lax.fori_loop(0, CH, addrow, 0)
        fire_write(c, b)

        @pl.when(c + 3 < nch)
        def _():
            fire_gather(c + 3, b)

    def outer(g, carry):
        c = 3 * g
        body_one(c, 0)

        @pl.when(c + 1 < nch)
        def _():
            body_one(c + 1, 1)

        @pl.when(c + 2 < nch)
        def _():
            body_one(c + 2, 2)

        return carry

    lax.fori_loop(0, (nch + 2) // 3, outer, 0)
    wait_write(0)
    wait_write(1)
    wait_write(2)
  return _gather_body


def _gather(pex, qex, row2d, col2d, nch):
    mesh = plsc.VectorSubcoreMesh(core_axis_name="c", subcore_axis_name="s")
    f = functools.partial(
        pl.kernel,
        mesh=mesh,
        out_type=jax.ShapeDtypeStruct((NWK * nch * CH, PW), jnp.float32),
        scratch_types=[
            pltpu.VMEM((nch, CH), jnp.int32),
            pltpu.VMEM((nch, CH), jnp.int32),
            pltpu.VMEM((CH, PW), jnp.float32),
            pltpu.VMEM((CH, PW), jnp.float32),
            pltpu.VMEM((CH, PW), jnp.float32),
            pltpu.VMEM((CH, PW), jnp.float32),
            pltpu.VMEM((CH, PW), jnp.float32),
            pltpu.VMEM((CH, PW), jnp.float32),
            pltpu.VMEM((CH, PW), jnp.float32),
            pltpu.VMEM((CH, PW), jnp.float32),
            pltpu.VMEM((CH, PW), jnp.float32),
            pltpu.SemaphoreType.DMA,
            pltpu.SemaphoreType.DMA,
            pltpu.SemaphoreType.DMA,
            pltpu.SemaphoreType.DMA,
            pltpu.SemaphoreType.DMA,
            pltpu.SemaphoreType.DMA,
        ],
    )(_make_gather_body(nch))
    return f(pex, qex, row2d, col2d)


def _edge_body(s_ref, ea_ref, w1d_ref, wd_ref, w2_ref, b2_ref,
               wc1_ref, bc1_ref, wc2_ref, bc2_ref, msk_ref, msk0_ref, ones_ref,
               m_ref):
    s = s_ref[...]
    zero = jnp.zeros_like(s)
    sel = jnp.where(msk_ref[...] > 0.5, s, zero)
    dist2 = lax.dot(sel * sel, ones_ref[...], precision=lax.Precision.DEFAULT)
    dist = jnp.sqrt(dist2)
    pre1 = (jnp.where(msk0_ref[...] > 0.5, s, zero) + dist * wd_ref[...]
            + lax.dot(ea_ref[...], w1d_ref[...], precision=lax.Precision.DEFAULT))
    h1 = _silu(pre1)
    msg = _silu(lax.dot(h1, w2_ref[...], precision=lax.Precision.DEFAULT) + b2_ref[...])
    t = _silu(lax.dot(msg, wc1_ref[...], precision=lax.Precision.DEFAULT) + bc1_ref[...])
    cw = lax.dot(t, wc2_ref[...], precision=lax.Precision.DEFAULT) + bc2_ref[...]
    m_ref[...] = msg + sel * cw


def _edge_mlp(s, ea, w1dp, wdp, w2p, b2p, wc1p, bc1p, wc2c, bc2r, msk, msk0,
              onescol, rows, be):
    return pl.pallas_call(
        _edge_body,
        grid=(rows // be,),
        in_specs=[
            pl.BlockSpec((be, PW), lambda i: (i, 0)),
            pl.BlockSpec((be, ED), lambda i: (i, 0)),
            pl.BlockSpec((ED, PW), lambda i: (0, 0)),
            pl.BlockSpec((1, PW), lambda i: (0, 0)),
            pl.BlockSpec((PW, PW), lambda i: (0, 0)),
            pl.BlockSpec((1, PW), lambda i: (0, 0)),
            pl.BlockSpec((PW, PW), lambda i: (0, 0)),
            pl.BlockSpec((1, PW), lambda i: (0, 0)),
            pl.BlockSpec((PW, 1), lambda i: (0, 0)),
            pl.BlockSpec((1, 1), lambda i: (0, 0)),
            pl.BlockSpec((1, PW), lambda i: (0, 0)),
            pl.BlockSpec((1, PW), lambda i: (0, 0)),
            pl.BlockSpec((PW, 1), lambda i: (0, 0)),
        ],
        out_specs=pl.BlockSpec((be, PW), lambda i: (i, 0)),
        out_shape=jax.ShapeDtypeStruct((rows, PW), jnp.float32),
    )(s, ea, w1dp, wdp, w2p, b2p, wc1p, bc1p, wc2c, bc2r, msk, msk0, onescol)


NSC = NPAD // NC
ACC_R = NSC + 8
DUMP = NSC
RPT_D = NSC // NS
EPT = E // NS
NCH_D = EPT // CH


def _make_scatter_body(nchd):
  def _scatter_body(m, row2d, out, idxr, mbuf0, mbuf1, mbuf2, mbuf3, acc,
                  seml0, seml1, seml2, seml3, sems0, sems1, sems2, sems3):
    cid = lax.axis_index("c")
    sid = lax.axis_index("s")
    lo = cid * NSC
    hi = lo + NSC
    mbuf = (mbuf0, mbuf1, mbuf2, mbuf3)
    seml = (seml0, seml1, seml2, seml3)
    sems = (sems0, sems1, sems2, sems3)
    zv = jnp.zeros((16,), jnp.float32)

    pltpu.sync_copy(row2d.at[sid], idxr)

    def remap(r, c2):
        for k in range(CH // 16):
            sl = pl.ds(k * 16, 16)
            v = idxr[r, sl]
            inr = (v >= lo) & (v < hi)
            idxr[r, sl] = jnp.where(inr, v - lo, DUMP)
        return c2

    lax.fori_loop(0, nchd, remap, 0)

    def zrow(r, c2):
        for k in range(PW // 16):
            mbuf0[r, pl.ds(k * 16, 16)] = zv
        return c2

    lax.fori_loop(0, CH, zrow, 0)

    def zcopy(j, c2):
        pltpu.sync_copy(mbuf0, acc.at[pl.ds(sid * RPT_D + j * CH, CH)])
        return c2

    lax.fori_loop(0, RPT_D // CH, zcopy, 0)
    plsc.subcore_barrier()

    base = sid * nchd * CH

    def fire_load(c, b):
        pltpu.async_copy(m.at[pl.ds(base + c * CH, CH)], mbuf[b], seml[b])

    def wait_load(b):
        pltpu.make_async_copy(m.at[pl.ds(base, CH)], mbuf[b], seml[b]).wait()

    def fire_scatter(c, b):
        pltpu.async_copy(mbuf[b], acc.at[idxr.at[c]], sems[b], add=True)

    def wait_scatter(b):
        pltpu.make_async_copy(mbuf[b], acc.at[idxr.at[0]], sems[b]).wait()

    fire_load(0, 0)
    fire_load(1, 1)

    def body_one(c, b, b2):
        wait_load(b)
        fire_scatter(c, b)

        @pl.when(c + 2 < nchd)
        def _():
            @pl.when(c >= 2)
            def _():
                wait_scatter(b2)

            fire_load(c + 2, b2)

    def outer(g, carry):
        c = 4 * g
        body_one(c, 0, 2)

        @pl.when(c + 1 < nchd)
        def _():
            body_one(c + 1, 1, 3)

        @pl.when(c + 2 < nchd)
        def _():
            body_one(c + 2, 2, 0)

        @pl.when(c + 3 < nchd)
        def _():
            body_one(c + 3, 3, 1)

        return carry

    lax.fori_loop(0, (nchd + 3) // 4, outer, 0)
    wait_scatter(0)
    wait_scatter(1)
    wait_scatter(2)
    wait_scatter(3)
    plsc.subcore_barrier()

    def ocopy(j, c2):
        pltpu.sync_copy(acc.at[pl.ds(sid * RPT_D + j * CH, CH)], mbuf0)
        pltpu.sync_copy(mbuf0, out.at[pl.ds(cid * NSC + sid * RPT_D + j * CH, CH)])
        return c2

    lax.fori_loop(0, RPT_D // CH, ocopy, 0)
  return _scatter_body


def _scatter(m, row2d, nchd):
    mesh = plsc.VectorSubcoreMesh(core_axis_name="c", subcore_axis_name="s")
    f = functools.partial(
        pl.kernel,
        mesh=mesh,
        out_type=jax.ShapeDtypeStruct((NPAD, PW), jnp.float32),
        scratch_types=[
            pltpu.VMEM((nchd, CH), jnp.int32),
            pltpu.VMEM((CH, PW), jnp.float32),
            pltpu.VMEM((CH, PW), jnp.float32),
            pltpu.VMEM((CH, PW), jnp.float32),
            pltpu.VMEM((CH, PW), jnp.float32),
            pltpu.VMEM_SHARED((ACC_R, PW), jnp.float32),
            pltpu.SemaphoreType.DMA,
            pltpu.SemaphoreType.DMA,
            pltpu.SemaphoreType.DMA,
            pltpu.SemaphoreType.DMA,
            pltpu.SemaphoreType.DMA,
            pltpu.SemaphoreType.DMA,
            pltpu.SemaphoreType.DMA,
            pltpu.SemaphoreType.DMA,
        ],
    )(_make_scatter_body(nchd))
    return f(m, row2d)


def _node_body(x_ref, agg_ref, agg2_ref, agg3_ref, agg4_ref, pos_ref,
               wn1a_ref, wn1b_ref, bn1_ref, wn2_ref, bn2_ref, xn_ref, pn_ref):
    aggf = ((agg_ref[...] + agg2_ref[...])
            + (agg3_ref[...] + agg4_ref[...]))
    agg = aggf[:, :H]
    coord = aggf[:, H:H + 3]
    h = _silu(lax.dot(x_ref[...], wn1a_ref[...], precision=_HI)
              + lax.dot(agg, wn1b_ref[...], precision=_HI) + bn1_ref[...])
    xn_ref[...] = lax.dot(h, wn2_ref[...], precision=_HI) + bn2_ref[...]
    pn_ref[...] = pos_ref[...] + coord


def _node_mlp(x, parts, parts2, parts3, parts4, pos, wn1a, wn1b, bn1r,
              wn2, bn2r):
    bn = 2000
    return pl.pallas_call(
        _node_body,
        grid=(N // bn,),
        in_specs=[
            pl.BlockSpec((bn, D), lambda i: (i, 0)),
            pl.BlockSpec((bn, PW), lambda i: (i, 0)),
            pl.BlockSpec((bn, PW), lambda i: (i, 0)),
            pl.BlockSpec((bn, PW), lambda i: (i, 0)),
            pl.BlockSpec((bn, PW), lambda i: (i, 0)),
            pl.BlockSpec((bn, 3), lambda i: (i, 0)),
            pl.BlockSpec((D, H), lambda i: (0, 0)),
            pl.BlockSpec((H, H), lambda i: (0, 0)),
            pl.BlockSpec((1, H), lambda i: (0, 0)),
            pl.BlockSpec((H, D), lambda i: (0, 0)),
            pl.BlockSpec((1, D), lambda i: (0, 0)),
        ],
        out_specs=[
            pl.BlockSpec((bn, D), lambda i: (i, 0)),
            pl.BlockSpec((bn, 3), lambda i: (i, 0)),
        ],
        out_shape=[
            jax.ShapeDtypeStruct((N, D), jnp.float32),
            jax.ShapeDtypeStruct((N, 3), jnp.float32),
        ],
    )(x, parts, parts2, parts3, parts4, pos, wn1a, wn1b, bn1r, wn2, bn2r)


def kernel(x, pos, edge_index, edge_attr, W1, b1, W2, b2,
           Wn1, bn1, Wn2, bn2, Wc1, bc1, Wc2, bc2):
    ncgs = (31, 31, 31, 32)
    row = edge_index[0]
    col = edge_index[1]
    sl_edges = [NWK * g * CH for g in ncgs]
    bounds = [0]
    for n_e in sl_edges:
        bounds.append(bounds[-1] + n_e)
    r3g, c3g, r3s, ncds = [], [], [], []
    for i, g in enumerate(ncgs):
        a, b = bounds[i], bounds[i + 1]
        r3g.append(row[a:b].reshape(NWK, g, CH))
        c3g.append(col[a:b].reshape(NWK, g, CH))
        nd = (b - a) // (NS * CH)
        ncds.append(nd)
        r3s.append(row[a:b].reshape(NS, nd, CH))
    w1a = W1[:D]
    w1b = W1[D:2 * D]
    b1r = b1.reshape(1, H)
    bn1r = bn1.reshape(1, H)
    bn2r = bn2.reshape(1, D)
    wn1a = Wn1[:D]
    wn1b = Wn1[D:]

    def padc(a):
        return jnp.pad(a, ((0, 0), (0, PW - a.shape[1])))

    def padr(a):
        return jnp.pad(a, ((0, PW - a.shape[0]), (0, 0)))

    wdp = padc(W1[2 * D:2 * D + 1])
    w1dp = padc(W1[2 * D + 1:])
    w2p = padr(padc(W2))
    b2p = padc(b2.reshape(1, H))
    wc1p = padr(padc(Wc1))
    bc1p = padc(bc1.reshape(1, H))
    wc2c = padr(Wc2)
    bc2r = bc2.reshape(1, 1)
    lane = jnp.arange(PW)
    msk = ((lane >= H) & (lane < H + 3)).astype(jnp.float32).reshape(1, PW)
    msk0 = (lane < H).astype(jnp.float32).reshape(1, PW)
    onescol = jnp.ones((PW, 1), jnp.float32)

    pex, qex = _prep(x, pos, w1a, w1b, b1r)
    aggs = []
    for i, g in enumerate(ncgs):
        a, b = bounds[i], bounds[i + 1]
        s_i = _gather(pex, qex, r3g[i], c3g[i], g)
        m_i = _edge_mlp(s_i, edge_attr[a:b], w1dp, wdp, w2p, b2p, wc1p, bc1p,
                        wc2c, bc2r, msk, msk0, onescol, b - a, (b - a) // 20)
        aggs.append(_scatter(m_i, r3s[i], ncds[i]))
    return _node_mlp(x, aggs[0], aggs[1], aggs[2], aggs[3], pos,
                     wn1a, wn1b, bn1r, Wn2, bn2r)
